# baseline (speedup 1.0000x reference)
import functools

import jax
import jax.numpy as jnp
from jax import lax
from jax.experimental import pallas as pl
from jax.experimental.pallas import tpu as pltpu
from jax.experimental.pallas import tpu_sc as plsc

_NN = 10000
_NE = 320000
_D = 128
_CH = 80
_NC = 2
_NS = 16
_NW = _NC * _NS
_U = _NW * _CH
_UNITS = (12, 30, 30, 30, 23)
_K = len(_UNITS)
_SLS = tuple(u * _U for u in _UNITS)
_OFFS = tuple(sum(_SLS[:k]) for k in range(_K))
_EB = 2560
_NNP = 10240
_RPT = _NNP // _NS


def _premul_body(n_ref, w_ref, p_ref, q_ref):
    n = n_ref[...]
    p_ref[...] = jnp.dot(n, w_ref[0:_D, :], preferred_element_type=jnp.float32)
    q_ref[...] = jnp.dot(n, w_ref[_D:2 * _D, :], preferred_element_type=jnp.float32)


def _premul(node, w1ab):
    b = 2000
    return pl.pallas_call(
        _premul_body,
        grid=(_NN // b,),
        in_specs=[pl.BlockSpec((b, _D), lambda i: (i, 0)),
                  pl.BlockSpec((2 * _D, _D), lambda i: (0, 0))],
        out_specs=[pl.BlockSpec((b, _D), lambda i: (i, 0)),
                   pl.BlockSpec((b, _D), lambda i: (i, 0))],
        out_shape=[jax.ShapeDtypeStruct((_NN, _D), jnp.float32),
                   jax.ShapeDtypeStruct((_NN, _D), jnp.float32)],
    )(node, w1ab)


def _edge_body(g_ref, e_ref, w1c_ref, b1_ref, w2_ref, b2_ref,
               lg_ref, lb_ref, *rest):
    u_ref, ne_ref = rest[-2], rest[-1]
    e = e_ref[...]
    x = (g_ref[...] + b1_ref[...]
         + jnp.dot(e, w1c_ref[...], preferred_element_type=jnp.float32))
    h = jnp.maximum(x, 0.0)
    o = jnp.dot(h, w2_ref[...], preferred_element_type=jnp.float32) + b2_ref[...]
    mu = jnp.mean(o, axis=-1, keepdims=True)
    oc = o - mu
    var = jnp.mean(oc * oc, axis=-1, keepdims=True)
    u = oc * lax.rsqrt(var + 1e-5) * lg_ref[...] + lb_ref[...]
    u_ref[...] = u
    ne_ref[...] = u + e


def _edge_mlp_slice(g, e_feat, w1c, b1, w2, b2, ln_g, ln_b, ks, ne_alias):
    base = _OFFS[ks] // _EB
    nblk = _SLS[ks] // _EB
    row_l = lambda i: (i, 0)
    row_g = lambda i, base=base: (i + base, 0)
    rep = lambda i: (0, 0)
    ins = [g, e_feat, w1c, b1, w2, b2, ln_g, ln_b]
    in_specs = [pl.BlockSpec((_EB, _D), row_l),
                pl.BlockSpec((_EB, _D), row_g),
                pl.BlockSpec((_D, _D), rep),
                pl.BlockSpec((1, _D), rep),
                pl.BlockSpec((_D, _D), rep),
                pl.BlockSpec((1, _D), rep),
                pl.BlockSpec((1, _D), rep),
                pl.BlockSpec((1, _D), rep)]
    aliases = {}
    if ne_alias is not None:
        ins.append(ne_alias)
        in_specs.append(pl.BlockSpec(memory_space=pl.ANY))
        aliases = {8: 1}
    return pl.pallas_call(
        _edge_body,
        grid=(nblk,),
        in_specs=in_specs,
        out_specs=[pl.BlockSpec((_EB, _D), row_l),
                   pl.BlockSpec((_EB, _D), row_g)],
        out_shape=[jax.ShapeDtypeStruct((_SLS[ks], _D), jnp.float32),
                   jax.ShapeDtypeStruct((_NE, _D), jnp.float32)],
        input_output_aliases=aliases,
    )(*ins)


def _pipe(nch, start, finish):
    start(0, 0)

    def body(kk, carry):
        c0 = 2 * kk
        start(c0 + 1, 1)
        finish(c0, 0)
        start(c0 + 2, 0)
        finish(c0 + 1, 1)
        return carry

    if nch % 2 == 1:
        lax.fori_loop(0, (nch - 1) // 2, body, 0)
        finish(nch - 1, 0)
    else:
        lax.fori_loop(0, (nch - 2) // 2, body, 0)
        start(nch - 1, 1)
        finish(nch - 2, 0)
        finish(nch - 1, 1)


def _vadd_into(ba, bb):
    def vrow(r, carry):
        for j in range(_D // 16):
            sl = pl.ds(j * 16, 16)
            ba[r, sl] = ba[r, sl] + bb[r, sl]
        return carry
    lax.fori_loop(0, _CH, vrow, 0)


def _sc_gather_slice(p, q, s_idx, r_idx, ks):
    mesh = plsc.VectorSubcoreMesh(core_axis_name="c", subcore_axis_name="s")
    nch = _UNITS[ks]
    epw = nch * _CH

    @functools.partial(
        pl.kernel, mesh=mesh,
        out_type=jax.ShapeDtypeStruct((_SLS[ks], _D), jnp.float32),
        scratch_types=[pltpu.VMEM((_CH,), jnp.int32),
                       pltpu.VMEM((_CH,), jnp.int32),
                       pltpu.VMEM((_CH,), jnp.int32),
                       pltpu.VMEM((_CH,), jnp.int32),
                       pltpu.VMEM((_CH, _D), jnp.float32),
                       pltpu.VMEM((_CH, _D), jnp.float32),
                       pltpu.VMEM((_CH, _D), jnp.float32),
                       pltpu.VMEM((_CH, _D), jnp.float32),
                       pltpu.SemaphoreType.DMA,
                       pltpu.SemaphoreType.DMA],
    )
    def k(p_hbm, q_hbm, s_hbm, r_hbm, g_hbm,
          si0, ri0, si1, ri1, ba0, bb0, ba1, bb1, sem0, sem1):
        wid = lax.axis_index("s") * _NC + lax.axis_index("c")
        ibase = _OFFS[ks] + wid * epw
        obase = wid * epw
        slots = ((si0, ri0, ba0, bb0, sem0),
                 (si1, ri1, ba1, bb1, sem1))

        def start(chunk, slot):
            sis, ris, ba, bb, sem = slots[slot]
            off = ibase + chunk * _CH
            pltpu.sync_copy(s_hbm.at[pl.ds(off, _CH)], sis)
            pltpu.sync_copy(r_hbm.at[pl.ds(off, _CH)], ris)
            pltpu.async_copy(p_hbm.at[sis], ba, sem)
            pltpu.async_copy(q_hbm.at[ris], bb, sem)

        def finish(chunk, slot):
            sis, ris, ba, bb, sem = slots[slot]
            pltpu.make_async_copy(p_hbm.at[sis], ba, sem).wait()
            pltpu.make_async_copy(q_hbm.at[ris], bb, sem).wait()
            _vadd_into(ba, bb)
            pltpu.sync_copy(ba, g_hbm.at[pl.ds(obase + chunk * _CH, _CH)])

        _pipe(nch, start, finish)

    return k(p, q, s_idx, r_idx)


def _sc_scatter(upds, r_idx, ks0):
    mesh = plsc.VectorSubcoreMesh(core_axis_name="c", subcore_axis_name="s")

    @functools.partial(
        pl.kernel, mesh=mesh,
        out_type=jax.ShapeDtypeStruct((_NC, _NNP, _D), jnp.float32),
        scratch_types=[pltpu.VMEM((_CH,), jnp.int32),
                       pltpu.VMEM((_CH,), jnp.int32),
                       pltpu.VMEM((_CH, _D), jnp.float32),
                       pltpu.VMEM((_CH, _D), jnp.float32),
                       pltpu.SemaphoreType.DMA,
                       pltpu.SemaphoreType.DMA,
                       pltpu.VMEM_SHARED((_NNP, _D), jnp.float32)],
    )
    def k(*refs):
        u_hbms = refs[:len(upds)]
        (r_hbm, o_hbm, ri0, ri1, buf0, buf1,
         sem0, sem1, agg_sh) = refs[len(upds):]
        c = lax.axis_index("c")
        s = lax.axis_index("s")
        wid = s * _NC + c

        def zrow(r, carry):
            for j in range(_D // 16):
                buf0[r, pl.ds(j * 16, 16)] = jnp.zeros((16,), jnp.float32)
            return carry
        lax.fori_loop(0, _CH, zrow, 0)
        for t in range(_RPT // _CH):
            pltpu.sync_copy(buf0, agg_sh.at[pl.ds(s * _RPT + t * _CH, _CH)])
        plsc.subcore_barrier()

        slots = ((ri0, buf0, sem0), (ri1, buf1, sem1))
        for ku, u_hbm in enumerate(u_hbms):
            ks = ks0 + ku
            epw = _UNITS[ks] * _CH
            ibase = _OFFS[ks] + wid * epw
            ubase = wid * epw

            def start(chunk, slot, u_hbm=u_hbm, ibase=ibase, ubase=ubase):
                ri, buf, sem = slots[slot]
                pltpu.sync_copy(r_hbm.at[pl.ds(ibase + chunk * _CH, _CH)], ri)
                pltpu.async_copy(u_hbm.at[pl.ds(ubase + chunk * _CH, _CH)],
                                 buf, sem)

            def finish(chunk, slot, u_hbm=u_hbm, ubase=ubase):
                ri, buf, sem = slots[slot]
                pltpu.make_async_copy(u_hbm.at[pl.ds(ubase, _CH)],
                                      buf, sem).wait()
                pltpu.sync_copy(buf, agg_sh.at[ri], add=True)

            _pipe(_UNITS[ks], start, finish)

        plsc.subcore_barrier()
        pltpu.sync_copy(agg_sh.at[pl.ds(s * _RPT, _RPT)],
                        o_hbm.at[c, pl.ds(s * _RPT, _RPT)])

    return k(*upds, r_idx)


def _node_body4(n_ref, *rest):
    aggs = rest[:-7]
    w1_ref, b1_ref, w2_ref, b2_ref, lg_ref, lb_ref, o_ref = rest[-7:]
    n = n_ref[...]
    a = aggs[0][0, :, :] + aggs[0][1, :, :]
    for ar in aggs[1:]:
        a = a + ar[0, :, :] + ar[1, :, :]
    x = (jnp.dot(n, w1_ref[0:_D, :], preferred_element_type=jnp.float32)
         + jnp.dot(a, w1_ref[_D:2 * _D, :], preferred_element_type=jnp.float32)
         + b1_ref[...])
    h = jnp.maximum(x, 0.0)
    o = jnp.dot(h, w2_ref[...], preferred_element_type=jnp.float32) + b2_ref[...]
    mu = jnp.mean(o, axis=-1, keepdims=True)
    oc = o - mu
    var = jnp.mean(oc * oc, axis=-1, keepdims=True)
    o_ref[...] = oc * lax.rsqrt(var + 1e-5) * lg_ref[...] + lb_ref[...] + n


def _node_mlp4(node, aggs, w1, b1, w2, b2, ln_g, ln_b):
    b = 2000
    rep = lambda i: (0, 0)
    agg_spec = pl.BlockSpec((2, b, _D), lambda i: (0, i, 0))
    return pl.pallas_call(
        _node_body4,
        grid=(_NN // b,),
        in_specs=[pl.BlockSpec((b, _D), lambda i: (i, 0))]
                 + [agg_spec] * len(aggs)
                 + [pl.BlockSpec((2 * _D, _D), rep),
                    pl.BlockSpec((1, _D), rep),
                    pl.BlockSpec((_D, _D), rep),
                    pl.BlockSpec((1, _D), rep),
                    pl.BlockSpec((1, _D), rep),
                    pl.BlockSpec((1, _D), rep)],
        out_specs=pl.BlockSpec((b, _D), lambda i: (i, 0)),
        out_shape=jax.ShapeDtypeStruct((_NN, _D), jnp.float32),
    )(node, *aggs, w1, b1, w2, b2, ln_g, ln_b)


def kernel(node_features, mesh_edge_features, senders, receivers,
           edge_params, node_params):
    senders = senders.astype(jnp.int32)
    receivers = receivers.astype(jnp.int32)
    w1e = edge_params['w1']
    row = lambda v: v.reshape(1, _D)

    p, q = _premul(node_features, w1e[:2 * _D])

    upds = []
    new_edge = None
    for ks in range(_K):
        g = _sc_gather_slice(p, q, senders, receivers, ks)
        upd_k, new_edge = _edge_mlp_slice(
            g, mesh_edge_features, w1e[2 * _D:],
            row(edge_params['b1']), edge_params['w2'], row(edge_params['b2']),
            row(edge_params['ln_g']), row(edge_params['ln_b']),
            ks, new_edge)
        upds.append(upd_k)

    aggs = [_sc_scatter(upds[:3], receivers, 0),
            _sc_scatter(upds[3:4], receivers, 3),
            _sc_scatter(upds[4:], receivers, 4)]
    new_node = _node_mlp4(
        node_features, aggs, node_params['w1'], row(node_params['b1']),
        node_params['w2'], row(node_params['b2']),
        row(node_params['ln_g']), row(node_params['ln_b']))
    return new_node, new_edge
